# Initial kernel scaffold; baseline (speedup 1.0000x reference)
import jax
import jax.numpy as jnp
from jax.experimental import pallas as pl


def kernel(x, edge_index, W1, b1, W2, b2):
    raise NotImplementedError("write your pallas kernel here")


# speedup vs baseline: 21.5848x; 21.5848x over previous
import functools

import jax
import jax.numpy as jnp
from jax import lax
from jax.experimental import pallas as pl
from jax.experimental.pallas import tpu as pltpu
from jax.experimental.pallas import tpu_sc as plsc

N = 10000
E = 320000
D_IN = 128
D_HID = 128
D_OUT = 64

NC = 2
NS = 16
CHUNK = 80
EROWS = E // CHUNK
EROWS_PER_TILE = EROWS // (NC * NS)
RPT = N // NS
ZROWS = 125

_mesh = plsc.VectorSubcoreMesh(core_axis_name="c", subcore_axis_name="s")
_sc_params = pltpu.CompilerParams(use_tc_tiling_on_sc=False)


def _zero_fill(zb, rows, width):
    @pl.loop(0, rows)
    def _(i):
        for t in range(width // 16):
            zb[i, pl.ds(t * 16, 16)] = jnp.zeros((16,), jnp.float32)


@functools.partial(
    pl.kernel,
    out_type=jax.ShapeDtypeStruct((NC, N, 16), jnp.float32),
    mesh=_mesh,
    scratch_types=[
        pltpu.VMEM((EROWS_PER_TILE, CHUNK), jnp.int32),
        pltpu.VMEM((CHUNK, 16), jnp.float32),
        pltpu.VMEM((ZROWS, 16), jnp.float32),
        pltpu.VMEM_SHARED((N, 16), jnp.float32),
    ],
    compiler_params=_sc_params,
)
def _sc_degree(dst_hbm, cnt_hbm, idx_d, ones_v, zb, acc):
    c = lax.axis_index("c")
    s = lax.axis_index("s")

    @pl.loop(0, CHUNK)
    def _(i):
        ones_v[i, :] = jnp.full((16,), 1.0, jnp.float32)

    _zero_fill(zb, ZROWS, 16)
    for k in range(RPT // ZROWS):
        pltpu.sync_copy(zb, acc.at[pl.ds(s * RPT + k * ZROWS, ZROWS)])

    row0 = (c * NS + s) * EROWS_PER_TILE
    pltpu.sync_copy(dst_hbm.at[pl.ds(row0, EROWS_PER_TILE)], idx_d)
    plsc.subcore_barrier()

    @pl.loop(0, EROWS_PER_TILE)
    def _(j):
        pltpu.sync_copy(ones_v, acc.at[idx_d.at[j]], add=True)

    plsc.subcore_barrier()
    pltpu.sync_copy(acc.at[pl.ds(s * RPT, RPT)], cnt_hbm.at[c, pl.ds(s * RPT, RPT)])


def _make_sc_scatter(D):
    @functools.partial(
        pl.kernel,
        out_type=jax.ShapeDtypeStruct((NC, N, D), jnp.float32),
        mesh=_mesh,
        scratch_types=[
            pltpu.VMEM((EROWS_PER_TILE, CHUNK), jnp.int32),
            pltpu.VMEM((EROWS_PER_TILE, CHUNK), jnp.int32),
            pltpu.VMEM((CHUNK, D), jnp.float32),
            pltpu.VMEM((ZROWS, D), jnp.float32),
            pltpu.VMEM_SHARED((N, D), jnp.float32),
            pltpu.SemaphoreType.DMA,
        ],
        compiler_params=_sc_params,
    )
    def _scat(src_hbm, dst_hbm, y_hbm, out_hbm, idx_s, idx_d, rows, zb, acc, sem):
        c = lax.axis_index("c")
        s = lax.axis_index("s")

        _zero_fill(zb, ZROWS, D)
        for k in range(RPT // ZROWS):
            pltpu.sync_copy(zb, acc.at[pl.ds(s * RPT + k * ZROWS, ZROWS)])

        row0 = (c * NS + s) * EROWS_PER_TILE
        pltpu.sync_copy(src_hbm.at[pl.ds(row0, EROWS_PER_TILE)], idx_s)
        pltpu.sync_copy(dst_hbm.at[pl.ds(row0, EROWS_PER_TILE)], idx_d)
        plsc.subcore_barrier()

        @pl.loop(0, EROWS_PER_TILE)
        def _(j):
            pltpu.async_copy(y_hbm.at[idx_s.at[j]], rows, sem).wait()
            pltpu.sync_copy(rows, acc.at[idx_d.at[j]], add=True)

        plsc.subcore_barrier()
        pltpu.sync_copy(acc.at[pl.ds(s * RPT, RPT)],
                        out_hbm.at[c, pl.ds(s * RPT, RPT)])

    return _scat


_sc_scatter_hid = _make_sc_scatter(D_HID)
_sc_scatter_out = _make_sc_scatter(D_OUT)

_B = 1000


def _dinv_from_cnt(cnt_blk):
    deg = cnt_blk[0, :, 0:1] + cnt_blk[1, :, 0:1] + 1.0
    return lax.rsqrt(deg)


def _tc1_body(x_ref, w1_ref, cnt_ref, y1_ref):
    dinv = _dinv_from_cnt(cnt_ref)
    xw = jnp.dot(x_ref[...], w1_ref[...], preferred_element_type=jnp.float32)
    y1_ref[...] = xw * dinv


def _tc2_body(acc_ref, y1_ref, cnt_ref, b1_ref, w2_ref, y2_ref):
    dinv = _dinv_from_cnt(cnt_ref)
    pre = (acc_ref[0] + acc_ref[1] + y1_ref[...]) * dinv + b1_ref[...]
    h = jnp.maximum(pre, 0.0)
    hw = jnp.dot(h, w2_ref[...], preferred_element_type=jnp.float32)
    y2_ref[...] = hw * dinv


def _tc3_body(acc_ref, y2_ref, cnt_ref, b2_ref, out_ref):
    dinv = _dinv_from_cnt(cnt_ref)
    out_ref[...] = (acc_ref[0] + acc_ref[1] + y2_ref[...]) * dinv + b2_ref[...]


def _tc1(x, W1, cnt):
    return pl.pallas_call(
        _tc1_body,
        grid=(N // _B,),
        in_specs=[
            pl.BlockSpec((_B, D_IN), lambda i: (i, 0)),
            pl.BlockSpec((D_IN, D_HID), lambda i: (0, 0)),
            pl.BlockSpec((NC, _B, 16), lambda i: (0, i, 0)),
        ],
        out_specs=pl.BlockSpec((_B, D_HID), lambda i: (i, 0)),
        out_shape=jax.ShapeDtypeStruct((N, D_HID), jnp.float32),
    )(x, W1, cnt)


def _tc2(acc1, y1, cnt, b1, W2):
    return pl.pallas_call(
        _tc2_body,
        grid=(N // _B,),
        in_specs=[
            pl.BlockSpec((NC, _B, D_HID), lambda i: (0, i, 0)),
            pl.BlockSpec((_B, D_HID), lambda i: (i, 0)),
            pl.BlockSpec((NC, _B, 16), lambda i: (0, i, 0)),
            pl.BlockSpec((1, D_HID), lambda i: (0, 0)),
            pl.BlockSpec((D_HID, D_OUT), lambda i: (0, 0)),
        ],
        out_specs=pl.BlockSpec((_B, D_OUT), lambda i: (i, 0)),
        out_shape=jax.ShapeDtypeStruct((N, D_OUT), jnp.float32),
    )(acc1, y1, cnt, b1, W2)


def _tc3(acc2, y2, cnt, b2):
    return pl.pallas_call(
        _tc3_body,
        grid=(N // _B,),
        in_specs=[
            pl.BlockSpec((NC, _B, D_OUT), lambda i: (0, i, 0)),
            pl.BlockSpec((_B, D_OUT), lambda i: (i, 0)),
            pl.BlockSpec((NC, _B, 16), lambda i: (0, i, 0)),
            pl.BlockSpec((1, D_OUT), lambda i: (0, 0)),
        ],
        out_specs=pl.BlockSpec((_B, D_OUT), lambda i: (i, 0)),
        out_shape=jax.ShapeDtypeStruct((N, D_OUT), jnp.float32),
    )(acc2, y2, cnt, b2)


def kernel(x, edge_index, W1, b1, W2, b2):
    src = edge_index[0].astype(jnp.int32).reshape(EROWS, CHUNK)
    dst = edge_index[1].astype(jnp.int32).reshape(EROWS, CHUNK)
    cnt = _sc_degree(dst)
    y1 = _tc1(x, W1, cnt)
    acc1 = _sc_scatter_hid(src, dst, y1)
    y2 = _tc2(acc1, y1, cnt, b1.reshape(1, D_HID), W2)
    acc2 = _sc_scatter_out(src, dst, y2)
    return _tc3(acc2, y2, cnt, b2.reshape(1, D_OUT))


# speedup vs baseline: 26.4046x; 1.2233x over previous
import functools

import jax
import jax.numpy as jnp
from jax import lax
from jax.experimental import pallas as pl
from jax.experimental.pallas import tpu as pltpu
from jax.experimental.pallas import tpu_sc as plsc

N = 10000
E = 320000
D_IN = 128
D_HID = 128
D_OUT = 64

NC = 2
NS = 16
CHUNK = 80
EROWS = E // CHUNK
EROWS_PER_TILE = EROWS // (NC * NS)
RPT = N // NS
ZROWS = 125

_mesh = plsc.VectorSubcoreMesh(core_axis_name="c", subcore_axis_name="s")
_sc_params = pltpu.CompilerParams(use_tc_tiling_on_sc=False)


def _zero_fill(zb, rows, width):
    @pl.loop(0, rows)
    def _(i):
        for t in range(width // 16):
            zb[i, pl.ds(t * 16, 16)] = jnp.zeros((16,), jnp.float32)


@functools.partial(
    pl.kernel,
    out_type=jax.ShapeDtypeStruct((NC, N, 16), jnp.float32),
    mesh=_mesh,
    scratch_types=[
        pltpu.VMEM((EROWS_PER_TILE, CHUNK), jnp.int32),
        pltpu.VMEM((CHUNK, 16), jnp.float32),
        pltpu.VMEM((ZROWS, 16), jnp.float32),
        pltpu.VMEM_SHARED((N, 16), jnp.float32),
    ],
    compiler_params=_sc_params,
)
def _sc_degree(dst_hbm, cnt_hbm, idx_d, ones_v, zb, acc):
    c = lax.axis_index("c")
    s = lax.axis_index("s")

    @pl.loop(0, CHUNK)
    def _(i):
        ones_v[i, :] = jnp.full((16,), 1.0, jnp.float32)

    _zero_fill(zb, ZROWS, 16)
    for k in range(RPT // ZROWS):
        pltpu.sync_copy(zb, acc.at[pl.ds(s * RPT + k * ZROWS, ZROWS)])

    row0 = (c * NS + s) * EROWS_PER_TILE
    pltpu.sync_copy(dst_hbm.at[pl.ds(row0, EROWS_PER_TILE)], idx_d)
    plsc.subcore_barrier()

    @pl.loop(0, EROWS_PER_TILE)
    def _(j):
        pltpu.sync_copy(ones_v, acc.at[idx_d.at[j]], add=True)

    plsc.subcore_barrier()
    pltpu.sync_copy(acc.at[pl.ds(s * RPT, RPT)], cnt_hbm.at[c, pl.ds(s * RPT, RPT)])


def _make_sc_scatter(npass):
    W = 64
    outs = [jax.ShapeDtypeStruct((NC, N, W), jnp.float32) for _ in range(npass)]

    @functools.partial(
        pl.kernel,
        out_type=outs,
        mesh=_mesh,
        scratch_types=[
            pltpu.VMEM((EROWS_PER_TILE, CHUNK), jnp.int32),
            pltpu.VMEM((EROWS_PER_TILE, CHUNK), jnp.int32),
            pltpu.VMEM((CHUNK, W), jnp.float32),
            pltpu.VMEM((CHUNK, W), jnp.float32),
            pltpu.VMEM((ZROWS, W), jnp.float32),
            pltpu.VMEM_SHARED((N, W), jnp.float32),
            pltpu.SemaphoreType.DMA,
            pltpu.SemaphoreType.DMA,
        ],
        compiler_params=_sc_params,
    )
    def _scat(src_hbm, dst_hbm, y_hbm, *rest):
        out_refs = rest[:npass]
        idx_s, idx_d, buf_a, buf_b, zb, acc, sem_a, sem_b = rest[npass:]
        c = lax.axis_index("c")
        s = lax.axis_index("s")

        _zero_fill(zb, ZROWS, W)

        row0 = (c * NS + s) * EROWS_PER_TILE
        pltpu.sync_copy(src_hbm.at[pl.ds(row0, EROWS_PER_TILE)], idx_s)
        pltpu.sync_copy(dst_hbm.at[pl.ds(row0, EROWS_PER_TILE)], idx_d)

        for h in range(npass):
            tab = y_hbm.at[h]
            for k in range(RPT // ZROWS):
                pltpu.sync_copy(zb, acc.at[pl.ds(s * RPT + k * ZROWS, ZROWS)])
            plsc.subcore_barrier()

            pltpu.async_copy(tab.at[idx_s.at[0]], buf_a, sem_a)

            @pl.loop(0, EROWS_PER_TILE - 1, step=2)
            def _(j):
                pltpu.async_copy(tab.at[idx_s.at[j + 1]], buf_b, sem_b)
                pltpu.make_async_copy(tab.at[idx_s.at[j]], buf_a, sem_a).wait()
                pltpu.sync_copy(buf_a, acc.at[idx_d.at[j]], add=True)
                pltpu.async_copy(tab.at[idx_s.at[j + 2]], buf_a, sem_a)
                pltpu.make_async_copy(tab.at[idx_s.at[j + 1]], buf_b, sem_b).wait()
                pltpu.sync_copy(buf_b, acc.at[idx_d.at[j + 1]], add=True)

            last = EROWS_PER_TILE - 1
            pltpu.make_async_copy(tab.at[idx_s.at[last]], buf_a, sem_a).wait()
            pltpu.sync_copy(buf_a, acc.at[idx_d.at[last]], add=True)

            plsc.subcore_barrier()
            pltpu.sync_copy(acc.at[pl.ds(s * RPT, RPT)],
                            out_refs[h].at[c, pl.ds(s * RPT, RPT)])
            plsc.subcore_barrier()

    return _scat


_sc_scatter_l1 = _make_sc_scatter(2)
_sc_scatter_l2 = _make_sc_scatter(1)

_B = 1000


def _dinv_from_cnt(cnt_blk):
    deg = cnt_blk[0, :, 0:1] + cnt_blk[1, :, 0:1] + 1.0
    return lax.rsqrt(deg)


def _tc1_body(x_ref, w1_ref, cnt_ref, y1_ref):
    dinv = _dinv_from_cnt(cnt_ref)
    xw = jnp.dot(x_ref[...], w1_ref[...], preferred_element_type=jnp.float32)
    y1 = xw * dinv
    y1_ref[0] = y1[:, :64]
    y1_ref[1] = y1[:, 64:]


def _tc2_body(a0_ref, a1_ref, y1_ref, cnt_ref, b1_ref, w2_ref, y2_ref):
    dinv = _dinv_from_cnt(cnt_ref)
    h0 = (a0_ref[0] + a0_ref[1] + y1_ref[0]) * dinv
    h1 = (a1_ref[0] + a1_ref[1] + y1_ref[1]) * dinv
    pre = jnp.concatenate([h0, h1], axis=1) + b1_ref[...]
    h = jnp.maximum(pre, 0.0)
    hw = jnp.dot(h, w2_ref[...], preferred_element_type=jnp.float32)
    y2_ref[...] = hw * dinv


def _tc3_body(acc_ref, y2_ref, cnt_ref, b2_ref, out_ref):
    dinv = _dinv_from_cnt(cnt_ref)
    out_ref[...] = (acc_ref[0] + acc_ref[1] + y2_ref[...]) * dinv + b2_ref[...]


def _tc1(x, W1, cnt):
    return pl.pallas_call(
        _tc1_body,
        grid=(N // _B,),
        in_specs=[
            pl.BlockSpec((_B, D_IN), lambda i: (i, 0)),
            pl.BlockSpec((D_IN, D_HID), lambda i: (0, 0)),
            pl.BlockSpec((NC, _B, 16), lambda i: (0, i, 0)),
        ],
        out_specs=pl.BlockSpec((2, _B, 64), lambda i: (0, i, 0)),
        out_shape=jax.ShapeDtypeStruct((2, N, 64), jnp.float32),
    )(x, W1, cnt)


def _tc2(a0, a1, y1, cnt, b1, W2):
    return pl.pallas_call(
        _tc2_body,
        grid=(N // _B,),
        in_specs=[
            pl.BlockSpec((NC, _B, 64), lambda i: (0, i, 0)),
            pl.BlockSpec((NC, _B, 64), lambda i: (0, i, 0)),
            pl.BlockSpec((2, _B, 64), lambda i: (0, i, 0)),
            pl.BlockSpec((NC, _B, 16), lambda i: (0, i, 0)),
            pl.BlockSpec((1, D_HID), lambda i: (0, 0)),
            pl.BlockSpec((D_HID, D_OUT), lambda i: (0, 0)),
        ],
        out_specs=pl.BlockSpec((_B, D_OUT), lambda i: (i, 0)),
        out_shape=jax.ShapeDtypeStruct((N, D_OUT), jnp.float32),
    )(a0, a1, y1, cnt, b1, W2)


def _tc3(acc2, y2, cnt, b2):
    return pl.pallas_call(
        _tc3_body,
        grid=(N // _B,),
        in_specs=[
            pl.BlockSpec((NC, _B, D_OUT), lambda i: (0, i, 0)),
            pl.BlockSpec((_B, D_OUT), lambda i: (i, 0)),
            pl.BlockSpec((NC, _B, 16), lambda i: (0, i, 0)),
            pl.BlockSpec((1, D_OUT), lambda i: (0, 0)),
        ],
        out_specs=pl.BlockSpec((_B, D_OUT), lambda i: (i, 0)),
        out_shape=jax.ShapeDtypeStruct((N, D_OUT), jnp.float32),
    )(acc2, y2, cnt, b2)


def kernel(x, edge_index, W1, b1, W2, b2):
    src = edge_index[0].astype(jnp.int32).reshape(EROWS, CHUNK)
    dst = edge_index[1].astype(jnp.int32).reshape(EROWS, CHUNK)
    cnt = _sc_degree(dst)
    y1 = _tc1(x, W1, cnt)
    a0, a1 = _sc_scatter_l1(src, dst, y1)
    y2 = _tc2(a0, a1, y1, cnt, b1.reshape(1, D_HID), W2)
    (acc2,) = _sc_scatter_l2(src, dst, y2.reshape(1, N, D_OUT))
    return _tc3(acc2, y2, cnt, b2.reshape(1, D_OUT))


# speedup vs baseline: 32.6962x; 1.2383x over previous
import functools

import jax
import jax.numpy as jnp
from jax import lax
from jax.experimental import pallas as pl
from jax.experimental.pallas import tpu as pltpu
from jax.experimental.pallas import tpu_sc as plsc

N = 10000
E = 320000
D_IN = 128
D_HID = 128
D_OUT = 64

NC = 2
NS = 16
CHUNK = 500
EROWS = E // CHUNK
EROWS_PER_TILE = EROWS // (NC * NS)
RPT = N // NS
ZROWS = 125

_mesh = plsc.VectorSubcoreMesh(core_axis_name="c", subcore_axis_name="s")
_sc_params = pltpu.CompilerParams(use_tc_tiling_on_sc=False,
                                  needs_layout_passes=False)


def _zero_fill(zb, rows, width):
    @pl.loop(0, rows)
    def _(i):
        for t in range(width // 16):
            zb[i, pl.ds(t * 16, 16)] = jnp.zeros((16,), jnp.float32)


@functools.partial(
    pl.kernel,
    out_type=jax.ShapeDtypeStruct((NC, N, 16), jnp.float32),
    mesh=_mesh,
    scratch_types=[
        pltpu.VMEM((EROWS_PER_TILE, CHUNK), jnp.int32),
        pltpu.VMEM((512,), jnp.float32),
        pltpu.VMEM((2000,), jnp.float32),
        pltpu.VMEM((1000,), jnp.float32),
        pltpu.VMEM((1000, 16), jnp.float32),
        pltpu.VMEM_SHARED((N,), jnp.float32),
    ],
    compiler_params=_sc_params,
)
def _sc_degree(dst_hbm, cnt_hbm, idx_d, ones_v, z1, tmp, rep, acc):
    c = lax.axis_index("c")
    s = lax.axis_index("s")

    @pl.loop(0, 32)
    def _(i):
        ones_v[pl.ds(i * 16, 16)] = jnp.full((16,), 1.0, jnp.float32)

    @pl.loop(0, 125)
    def _(i):
        z1[pl.ds(i * 16, 16)] = jnp.zeros((16,), jnp.float32)

    @pl.when(s == 0)
    def _():
        for k in range(5):
            pltpu.sync_copy(z1, acc.at[pl.ds(k * 2000, 2000)])

    row0 = (c * NS + s) * EROWS_PER_TILE
    pltpu.sync_copy(dst_hbm.at[pl.ds(row0, EROWS_PER_TILE)], idx_d)
    plsc.subcore_barrier()

    @pl.loop(0, EROWS_PER_TILE)
    def _(j):
        pltpu.sync_copy(ones_v.at[pl.ds(0, CHUNK)], acc.at[idx_d.at[j]],
                        add=True)

    plsc.subcore_barrier()

    @pl.when(s < 10)
    def _():
        pltpu.sync_copy(acc.at[pl.ds(s * 1000, 1000)], tmp)

        @pl.loop(0, 1000)
        def _(r):
            idxv = jnp.zeros((16,), jnp.int32) + r
            rep[r, :] = plsc.load_gather(tmp, [idxv])

        pltpu.sync_copy(rep, cnt_hbm.at[c, pl.ds(s * 1000, 1000)])


def _make_sc_scatter(npass):
    W = 64
    outs = [jax.ShapeDtypeStruct((NC, N, W), jnp.float32) for _ in range(npass)]

    @functools.partial(
        pl.kernel,
        out_type=outs,
        mesh=_mesh,
        scratch_types=[
            pltpu.VMEM((EROWS_PER_TILE, CHUNK), jnp.int32),
            pltpu.VMEM((EROWS_PER_TILE, CHUNK), jnp.int32),
            pltpu.VMEM((CHUNK, W), jnp.float32),
            pltpu.VMEM((CHUNK, W), jnp.float32),
            pltpu.VMEM_SHARED((N, W), jnp.float32),
            pltpu.SemaphoreType.DMA,
            pltpu.SemaphoreType.DMA,
        ],
        compiler_params=_sc_params,
    )
    def _scat(src_hbm, dst_hbm, y_hbm, *rest):
        out_refs = rest[:npass]
        idx_s, idx_d, buf_a, buf_b, acc, sem_a, sem_b = rest[npass:]
        c = lax.axis_index("c")
        s = lax.axis_index("s")

        row0 = (c * NS + s) * EROWS_PER_TILE
        pltpu.sync_copy(src_hbm.at[pl.ds(row0, EROWS_PER_TILE)], idx_s)
        pltpu.sync_copy(dst_hbm.at[pl.ds(row0, EROWS_PER_TILE)], idx_d)

        for h in range(npass):
            tab = y_hbm.at[h]
            _zero_fill(buf_a, CHUNK, W)
            pltpu.sync_copy(buf_a, acc.at[pl.ds(s * RPT, CHUNK)])
            pltpu.sync_copy(buf_a.at[pl.ds(0, RPT - CHUNK)],
                            acc.at[pl.ds(s * RPT + CHUNK, RPT - CHUNK)])
            plsc.subcore_barrier()

            pltpu.async_copy(tab.at[idx_s.at[0]], buf_a, sem_a)

            @pl.loop(0, EROWS_PER_TILE - 2, step=2)
            def _(j):
                pltpu.async_copy(tab.at[idx_s.at[j + 1]], buf_b, sem_b)
                pltpu.make_async_copy(tab.at[idx_s.at[j]], buf_a, sem_a).wait()
                pltpu.sync_copy(buf_a, acc.at[idx_d.at[j]], add=True)
                pltpu.async_copy(tab.at[idx_s.at[j + 2]], buf_a, sem_a)
                pltpu.make_async_copy(tab.at[idx_s.at[j + 1]], buf_b,
                                      sem_b).wait()
                pltpu.sync_copy(buf_b, acc.at[idx_d.at[j + 1]], add=True)

            last = EROWS_PER_TILE - 2
            pltpu.async_copy(tab.at[idx_s.at[last + 1]], buf_b, sem_b)
            pltpu.make_async_copy(tab.at[idx_s.at[last]], buf_a, sem_a).wait()
            pltpu.sync_copy(buf_a, acc.at[idx_d.at[last]], add=True)
            pltpu.make_async_copy(tab.at[idx_s.at[last + 1]], buf_b,
                                  sem_b).wait()
            pltpu.sync_copy(buf_b, acc.at[idx_d.at[last + 1]], add=True)

            plsc.subcore_barrier()
            pltpu.sync_copy(acc.at[pl.ds(s * RPT, RPT)],
                            out_refs[h].at[c, pl.ds(s * RPT, RPT)])
            plsc.subcore_barrier()

    return _scat


_sc_scatter_l1 = _make_sc_scatter(2)
_sc_scatter_l2 = _make_sc_scatter(1)

_B = 1000


def _dinv_from_cnt(cnt_blk):
    deg = cnt_blk[0, :, 0:1] + cnt_blk[1, :, 0:1] + 1.0
    return lax.rsqrt(deg)


def _tc1_body(x_ref, w1_ref, cnt_ref, y1_ref):
    dinv = _dinv_from_cnt(cnt_ref)
    xw = jnp.dot(x_ref[...], w1_ref[...], preferred_element_type=jnp.float32)
    y1 = xw * dinv
    y1_ref[0] = y1[:, :64]
    y1_ref[1] = y1[:, 64:]


def _tc2_body(a0_ref, a1_ref, y1_ref, cnt_ref, b1_ref, w2_ref, y2_ref):
    dinv = _dinv_from_cnt(cnt_ref)
    h0 = (a0_ref[0] + a0_ref[1] + y1_ref[0]) * dinv
    h1 = (a1_ref[0] + a1_ref[1] + y1_ref[1]) * dinv
    pre = jnp.concatenate([h0, h1], axis=1) + b1_ref[...]
    h = jnp.maximum(pre, 0.0)
    hw = jnp.dot(h, w2_ref[...], preferred_element_type=jnp.float32)
    y2_ref[...] = hw * dinv


def _tc3_body(acc_ref, y2_ref, cnt_ref, b2_ref, out_ref):
    dinv = _dinv_from_cnt(cnt_ref)
    out_ref[...] = (acc_ref[0] + acc_ref[1] + y2_ref[...]) * dinv + b2_ref[...]


def _tc1(x, W1, cnt):
    return pl.pallas_call(
        _tc1_body,
        grid=(N // _B,),
        in_specs=[
            pl.BlockSpec((_B, D_IN), lambda i: (i, 0)),
            pl.BlockSpec((D_IN, D_HID), lambda i: (0, 0)),
            pl.BlockSpec((NC, _B, 16), lambda i: (0, i, 0)),
        ],
        out_specs=pl.BlockSpec((2, _B, 64), lambda i: (0, i, 0)),
        out_shape=jax.ShapeDtypeStruct((2, N, 64), jnp.float32),
    )(x, W1, cnt)


def _tc2(a0, a1, y1, cnt, b1, W2):
    return pl.pallas_call(
        _tc2_body,
        grid=(N // _B,),
        in_specs=[
            pl.BlockSpec((NC, _B, 64), lambda i: (0, i, 0)),
            pl.BlockSpec((NC, _B, 64), lambda i: (0, i, 0)),
            pl.BlockSpec((2, _B, 64), lambda i: (0, i, 0)),
            pl.BlockSpec((NC, _B, 16), lambda i: (0, i, 0)),
            pl.BlockSpec((1, D_HID), lambda i: (0, 0)),
            pl.BlockSpec((D_HID, D_OUT), lambda i: (0, 0)),
        ],
        out_specs=pl.BlockSpec((_B, D_OUT), lambda i: (i, 0)),
        out_shape=jax.ShapeDtypeStruct((N, D_OUT), jnp.float32),
    )(a0, a1, y1, cnt, b1, W2)


def _tc3(acc2, y2, cnt, b2):
    return pl.pallas_call(
        _tc3_body,
        grid=(N // _B,),
        in_specs=[
            pl.BlockSpec((NC, _B, D_OUT), lambda i: (0, i, 0)),
            pl.BlockSpec((_B, D_OUT), lambda i: (i, 0)),
            pl.BlockSpec((NC, _B, 16), lambda i: (0, i, 0)),
            pl.BlockSpec((1, D_OUT), lambda i: (0, 0)),
        ],
        out_specs=pl.BlockSpec((_B, D_OUT), lambda i: (i, 0)),
        out_shape=jax.ShapeDtypeStruct((N, D_OUT), jnp.float32),
    )(acc2, y2, cnt, b2)


def kernel(x, edge_index, W1, b1, W2, b2):
    src = edge_index[0].astype(jnp.int32).reshape(EROWS, CHUNK)
    dst = edge_index[1].astype(jnp.int32).reshape(EROWS, CHUNK)
    cnt = _sc_degree(dst)
    y1 = _tc1(x, W1, cnt)
    a0, a1 = _sc_scatter_l1(src, dst, y1)
    y2 = _tc2(a0, a1, y1, cnt, b1.reshape(1, D_HID), W2)
    (acc2,) = _sc_scatter_l2(src, dst, y2.reshape(1, N, D_OUT))
    return _tc3(acc2, y2, cnt, b2.reshape(1, D_OUT))
